# Initial kernel scaffold; baseline (speedup 1.0000x reference)
import jax
import jax.numpy as jnp
from jax.experimental import pallas as pl


def kernel(labels, src_ones):
    raise NotImplementedError("write your pallas kernel here")


# speedup vs baseline: 1.0462x; 1.0462x over previous
import functools

import jax
import jax.numpy as jnp
from jax import lax
from jax.experimental import pallas as pl
from jax.experimental.pallas import tpu as pltpu
from jax.experimental.pallas import tpu_sc as plsc

_B = 16384
_N = 1000
_NC = 2
_NS = 16
_NW = _NC * _NS
_RPW = _B // _NW
_CHUNK = 32
_NCHUNK = _RPW // _CHUNK
_CW = _CHUNK * _N


def _make_sc_one_hot():
    mesh = plsc.VectorSubcoreMesh(core_axis_name="c", subcore_axis_name="s")

    @functools.partial(
        pl.kernel,
        out_type=jax.ShapeDtypeStruct((_B * _N,), jnp.float32),
        mesh=mesh,
        compiler_params=pltpu.CompilerParams(needs_layout_passes=False),
        scratch_types=[
            pltpu.VMEM((_RPW,), jnp.int32),
            pltpu.VMEM((_RPW,), jnp.float32),
            pltpu.VMEM((_CW,), jnp.float32),
            pltpu.VMEM((_CW,), jnp.float32),
            pltpu.SemaphoreType.DMA,
            pltpu.SemaphoreType.DMA,
        ],
    )
    def one_hot_kernel(labels_hbm, src_hbm, out_hbm,
                       lab_v, src_v, buf0, buf1, sem0, sem1):
        wid = lax.axis_index("s") * _NC + lax.axis_index("c")
        row0 = wid * _RPW

        pltpu.sync_copy(labels_hbm.at[pl.ds(row0, _RPW)], lab_v)
        pltpu.sync_copy(src_hbm.at[pl.ds(row0, _RPW)], src_v)

        zeros16 = jnp.zeros((16,), jnp.float32)

        def zero_body(i, carry):
            base = i * 128
            for u in range(8):
                buf0[pl.ds(base + u * 16, 16)] = zeros16
                buf1[pl.ds(base + u * 16, 16)] = zeros16
            return carry

        lax.fori_loop(0, _CW // 128, zero_body, 0)

        iota16 = lax.iota(jnp.int32, 16)
        bufs = (buf0, buf1)
        sems = (sem0, sem1)
        handles = [None, None]

        def scatter_chunk(buf, c, values_from_src):
            for j in range(_CHUNK // 16):
                off = c * _CHUNK + j * 16
                lab16 = lab_v[pl.ds(off, 16)]
                idx = (iota16 + j * 16) * _N + lab16
                if values_from_src:
                    plsc.store_scatter(buf, [idx], src_v[pl.ds(off, 16)])
                else:
                    plsc.store_scatter(buf, [idx], zeros16)

        for c in range(_NCHUNK):
            b = c % 2
            buf = bufs[b]
            if c >= 2:
                handles[b].wait()
                scatter_chunk(buf, c - 2, values_from_src=False)
            scatter_chunk(buf, c, values_from_src=True)
            dst = out_hbm.at[pl.ds((row0 + c * _CHUNK) * _N, _CW)]
            handles[b] = pltpu.async_copy(buf, dst, sems[b])

        handles[0].wait()
        handles[1].wait()

    return one_hot_kernel


_sc_one_hot = _make_sc_one_hot()


def kernel(labels, src_ones):
    labels_flat = labels.reshape(_B).astype(jnp.int32)
    src_flat = src_ones.reshape(_B).astype(jnp.float32)
    out = _sc_one_hot(labels_flat, src_flat)
    return out.reshape(_B, _N)


# speedup vs baseline: 1.4950x; 1.4291x over previous
import functools

import jax
import jax.numpy as jnp
from jax import lax
from jax.experimental import pallas as pl
from jax.experimental.pallas import tpu as pltpu
from jax.experimental.pallas import tpu_sc as plsc

_B = 16384
_N = 1000
_NC = 2
_NS = 16
_NW = _NC * _NS
_RPW = _B // _NW
_CHUNK = 32
_NCHUNK = _RPW // _CHUNK


def _make_sc_one_hot():
    mesh = plsc.VectorSubcoreMesh(core_axis_name="c", subcore_axis_name="s")

    @functools.partial(
        pl.kernel,
        out_type=jax.ShapeDtypeStruct((_B, _N), jnp.float32),
        mesh=mesh,
        compiler_params=pltpu.CompilerParams(needs_layout_passes=False),
        scratch_types=[
            pltpu.VMEM((_RPW,), jnp.int32),
            pltpu.VMEM((_RPW,), jnp.float32),
            pltpu.VMEM((_CHUNK, _N), jnp.float32),
            pltpu.VMEM((_CHUNK, _N), jnp.float32),
            pltpu.SemaphoreType.DMA,
            pltpu.SemaphoreType.DMA,
        ],
    )
    def one_hot_kernel(labels_hbm, src_hbm, zeros_hbm, out_hbm,
                       lab_v, src_v, buf0, buf1, sem0, sem1):
        wid = lax.axis_index("s") * _NC + lax.axis_index("c")
        row0 = wid * _RPW

        pltpu.sync_copy(labels_hbm.at[pl.ds(row0, _RPW)], lab_v)
        pltpu.sync_copy(src_hbm.at[pl.ds(row0, _RPW)], src_v)
        pltpu.sync_copy(zeros_hbm, buf0)
        pltpu.sync_copy(zeros_hbm, buf1)

        zeros16 = jnp.zeros((16,), jnp.float32)
        iota16 = lax.iota(jnp.int32, 16)
        bufs = (buf0, buf1)
        sems = (sem0, sem1)
        handles = [None, None]

        def scatter_chunk(buf, c, values_from_src):
            for j in range(_CHUNK // 16):
                off = c * _CHUNK + j * 16
                lab16 = lab_v[pl.ds(off, 16)]
                row16 = iota16 + j * 16
                if values_from_src:
                    plsc.store_scatter(buf, [row16, lab16],
                                       src_v[pl.ds(off, 16)])
                else:
                    plsc.store_scatter(buf, [row16, lab16], zeros16)

        for c in range(_NCHUNK):
            b = c % 2
            buf = bufs[b]
            if c >= 2:
                handles[b].wait()
                scatter_chunk(buf, c - 2, values_from_src=False)
            scatter_chunk(buf, c, values_from_src=True)
            dst = out_hbm.at[pl.ds(row0 + c * _CHUNK, _CHUNK), :]
            handles[b] = pltpu.async_copy(buf, dst, sems[b])

        handles[0].wait()
        handles[1].wait()

    return one_hot_kernel


_sc_one_hot = _make_sc_one_hot()


def kernel(labels, src_ones):
    labels_flat = labels.reshape(_B).astype(jnp.int32)
    src_flat = src_ones.reshape(_B).astype(jnp.float32)
    zeros_chunk = jnp.zeros((_CHUNK, _N), jnp.float32)
    return _sc_one_hot(labels_flat, src_flat, zeros_chunk)


# speedup vs baseline: 3.1069x; 2.0781x over previous
import functools

import jax
import jax.numpy as jnp
from jax import lax
from jax.experimental import pallas as pl
from jax.experimental.pallas import tpu as pltpu
from jax.experimental.pallas import tpu_sc as plsc

_B = 16384
_N = 1000
_NC = 2
_NS = 16
_NW = _NC * _NS
_CPW = _B // _NW
_RCHUNK = 40
_NCHUNK = _N // _RCHUNK
_NGRP = _CPW // 16


def _make_sc_one_hot():
    mesh = plsc.VectorSubcoreMesh(core_axis_name="c", subcore_axis_name="s")

    @functools.partial(
        pl.kernel,
        out_type=jax.ShapeDtypeStruct((_N, _B), jnp.float32),
        mesh=mesh,
        compiler_params=pltpu.CompilerParams(needs_layout_passes=False),
        scratch_types=[
            pltpu.VMEM((_CPW,), jnp.int32),
            pltpu.VMEM((_CPW,), jnp.float32),
            pltpu.VMEM((_RCHUNK, _CPW), jnp.float32),
            pltpu.VMEM((_RCHUNK, _CPW), jnp.float32),
            pltpu.SemaphoreType.DMA,
            pltpu.SemaphoreType.DMA,
        ],
    )
    def one_hot_kernel(labels_hbm, src_hbm, zeros_hbm, out_hbm,
                       lab_v, src_v, buf0, buf1, sem0, sem1):
        wid = lax.axis_index("s") * _NC + lax.axis_index("c")
        col0 = wid * _CPW

        pltpu.sync_copy(labels_hbm.at[pl.ds(col0, _CPW)], lab_v)
        pltpu.sync_copy(src_hbm.at[pl.ds(col0, _CPW)], src_v)
        pltpu.sync_copy(zeros_hbm, buf0)
        pltpu.sync_copy(zeros_hbm, buf1)

        zeros16 = jnp.zeros((16,), jnp.float32)
        iota16 = lax.iota(jnp.int32, 16)
        bufs = (buf0, buf1)
        sems = (sem0, sem1)
        handles = [None, None]

        def scan_chunk(buf, new_r0, old_r0):
            def body(g, carry):
                lab16 = lab_v[pl.ds(g * 16, 16)]
                col16 = iota16 + g * 16
                if old_r0 is not None:
                    old_row = lab16 - old_r0
                    old_msk = (old_row >= 0) & (old_row < _RCHUNK)
                    plsc.store_scatter(buf, [old_row, col16], zeros16,
                                       mask=old_msk)
                new_row = lab16 - new_r0
                new_msk = (new_row >= 0) & (new_row < _RCHUNK)
                plsc.store_scatter(buf, [new_row, col16],
                                   src_v[pl.ds(g * 16, 16)], mask=new_msk)
                return carry

            lax.fori_loop(0, _NGRP, body, 0)

        for c in range(_NCHUNK):
            b = c % 2
            buf = bufs[b]
            if c >= 2:
                handles[b].wait()
                scan_chunk(buf, c * _RCHUNK, (c - 2) * _RCHUNK)
            else:
                scan_chunk(buf, c * _RCHUNK, None)
            dst = out_hbm.at[pl.ds(c * _RCHUNK, _RCHUNK), pl.ds(col0, _CPW)]
            handles[b] = pltpu.async_copy(buf, dst, sems[b])

        handles[0].wait()
        handles[1].wait()

    return one_hot_kernel


_sc_one_hot = _make_sc_one_hot()


def kernel(labels, src_ones):
    labels_flat = labels.reshape(_B).astype(jnp.int32)
    src_flat = src_ones.reshape(_B).astype(jnp.float32)
    zeros_chunk = jnp.zeros((_RCHUNK, _CPW), jnp.float32)
    out_t = _sc_one_hot(labels_flat, src_flat, zeros_chunk)
    return out_t.T


# speedup vs baseline: 3.1881x; 1.0261x over previous
import functools

import jax
import jax.numpy as jnp
from jax import lax
from jax.experimental import pallas as pl
from jax.experimental.pallas import tpu as pltpu
from jax.experimental.pallas import tpu_sc as plsc

_B = 16384
_N = 1000
_NC = 2
_NS = 16
_NW = _NC * _NS
_CPW = _B // _NW
_RCHUNK = 40
_NCHUNK = _N // _RCHUNK
_NGRP = _CPW // 16


def _make_sc_one_hot():
    mesh = plsc.VectorSubcoreMesh(core_axis_name="c", subcore_axis_name="s")

    @functools.partial(
        pl.kernel,
        out_type=jax.ShapeDtypeStruct((_N, _B), jnp.float32),
        mesh=mesh,
        compiler_params=pltpu.CompilerParams(needs_layout_passes=False),
        scratch_types=[
            pltpu.VMEM((_CPW,), jnp.int32),
            pltpu.VMEM((_CPW,), jnp.float32),
            pltpu.VMEM((_RCHUNK, _CPW), jnp.float32),
            pltpu.VMEM((_RCHUNK, _CPW), jnp.float32),
            pltpu.SemaphoreType.DMA,
            pltpu.SemaphoreType.DMA,
        ],
    )
    def one_hot_kernel(labels_hbm, src_hbm, zeros_hbm, out_hbm,
                       lab_v, src_v, buf0, buf1, sem0, sem1):
        wid = lax.axis_index("s") * _NC + lax.axis_index("c")
        col0 = wid * _CPW

        pltpu.sync_copy(labels_hbm.at[pl.ds(col0, _CPW)], lab_v)
        pltpu.sync_copy(src_hbm.at[pl.ds(col0, _CPW)], src_v)
        pltpu.sync_copy(zeros_hbm, buf0)
        pltpu.sync_copy(zeros_hbm, buf1)

        zeros16 = jnp.zeros((16,), jnp.float32)
        iota16 = lax.iota(jnp.int32, 16)
        bufs = (buf0, buf1)
        sems = (sem0, sem1)

        def scan_chunk(buf, new_r0, old_r0):
            def body(g, carry):
                lab16 = lab_v[pl.ds(g * 16, 16)]
                col16 = iota16 + g * 16
                if old_r0 is not None:
                    old_row = lab16 - old_r0
                    old_msk = (old_row >= 0) & (old_row < _RCHUNK)
                    plsc.store_scatter(buf, [old_row, col16], zeros16,
                                       mask=old_msk)
                new_row = lab16 - new_r0
                new_msk = (new_row >= 0) & (new_row < _RCHUNK)
                plsc.store_scatter(buf, [new_row, col16],
                                   src_v[pl.ds(g * 16, 16)], mask=new_msk)
                return carry

            lax.fori_loop(0, _NGRP, body, 0)

        def start_dma(buf, c, sem):
            dst = out_hbm.at[pl.ds(c * _RCHUNK, _RCHUNK), pl.ds(col0, _CPW)]
            pltpu.async_copy(buf, dst, sem)

        def wait_dma(buf, sem):
            pltpu.make_async_copy(
                buf, out_hbm.at[pl.ds(0, _RCHUNK), pl.ds(col0, _CPW)], sem
            ).wait()

        for b in range(2):
            scan_chunk(bufs[b], b * _RCHUNK, None)
            start_dma(bufs[b], b, sems[b])

        def round_body(g, carry):
            c0 = 2 + g * 2
            for b in range(2):
                c = c0 + b
                r0 = c * _RCHUNK
                wait_dma(bufs[b], sems[b])
                scan_chunk(bufs[b], r0, r0 - 2 * _RCHUNK)
                start_dma(bufs[b], c, sems[b])
            return carry

        lax.fori_loop(0, (_NCHUNK - 3) // 2, round_body, 0)

        c = _NCHUNK - 1
        wait_dma(buf0, sem0)
        scan_chunk(buf0, c * _RCHUNK, (c - 2) * _RCHUNK)
        start_dma(buf0, c, sem0)
        wait_dma(buf1, sem1)
        wait_dma(buf0, sem0)

    return one_hot_kernel


_sc_one_hot = _make_sc_one_hot()


def kernel(labels, src_ones):
    labels_flat = labels.reshape(_B).astype(jnp.int32)
    src_flat = src_ones.reshape(_B).astype(jnp.float32)
    zeros_chunk = jnp.zeros((_RCHUNK, _CPW), jnp.float32)
    out_t = _sc_one_hot(labels_flat, src_flat, zeros_chunk)
    return out_t.T


# speedup vs baseline: 3.2123x; 1.0076x over previous
import functools

import jax
import jax.numpy as jnp
from jax import lax
from jax.experimental import pallas as pl
from jax.experimental.pallas import tpu as pltpu
from jax.experimental.pallas import tpu_sc as plsc

_B = 16384
_N = 1000
_NC = 2
_NS = 16
_NW = _NC * _NS
_CPW = _B // _NW
_RCHUNK = 40
_NCHUNK = _N // _RCHUNK
_NGRP = _CPW // 16


def _make_sc_one_hot():
    mesh = plsc.VectorSubcoreMesh(core_axis_name="c", subcore_axis_name="s")

    @functools.partial(
        pl.kernel,
        out_type=jax.ShapeDtypeStruct((_N, _B), jnp.float32),
        mesh=mesh,
        compiler_params=pltpu.CompilerParams(
            needs_layout_passes=False,
            skip_device_barrier=True,
            disable_bounds_checks=True,
            disable_semaphore_checks=True,
        ),
        scratch_types=[
            pltpu.VMEM((_CPW,), jnp.int32),
            pltpu.VMEM((_CPW,), jnp.float32),
            pltpu.VMEM((_RCHUNK, _CPW), jnp.float32),
            pltpu.VMEM((_RCHUNK, _CPW), jnp.float32),
            pltpu.SemaphoreType.DMA,
            pltpu.SemaphoreType.DMA,
        ],
    )
    def one_hot_kernel(labels_hbm, src_hbm, zeros_hbm, out_hbm,
                       lab_v, src_v, buf0, buf1, sem0, sem1):
        wid = lax.axis_index("s") * _NC + lax.axis_index("c")
        col0 = wid * _CPW

        pltpu.sync_copy(labels_hbm.at[pl.ds(col0, _CPW)], lab_v)
        pltpu.sync_copy(src_hbm.at[pl.ds(col0, _CPW)], src_v)
        pltpu.sync_copy(zeros_hbm, buf0)
        pltpu.sync_copy(zeros_hbm, buf1)

        zeros16 = jnp.zeros((16,), jnp.float32)
        iota16 = lax.iota(jnp.int32, 16)
        bufs = (buf0, buf1)
        sems = (sem0, sem1)

        def scan_chunk(buf, new_r0, old_r0):
            def body(g, carry):
                lab16 = lab_v[pl.ds(g * 16, 16)]
                col16 = iota16 + g * 16
                if old_r0 is not None:
                    old_row = lab16 - old_r0
                    old_msk = (old_row >= 0) & (old_row < _RCHUNK)
                    plsc.store_scatter(buf, [old_row, col16], zeros16,
                                       mask=old_msk)
                new_row = lab16 - new_r0
                new_msk = (new_row >= 0) & (new_row < _RCHUNK)
                plsc.store_scatter(buf, [new_row, col16],
                                   src_v[pl.ds(g * 16, 16)], mask=new_msk)
                return carry

            lax.fori_loop(0, _NGRP, body, 0)

        def start_dma(buf, c, sem):
            dst = out_hbm.at[pl.ds(c * _RCHUNK, _RCHUNK), pl.ds(col0, _CPW)]
            pltpu.async_copy(buf, dst, sem)

        def wait_dma(buf, sem):
            pltpu.make_async_copy(
                buf, out_hbm.at[pl.ds(0, _RCHUNK), pl.ds(col0, _CPW)], sem
            ).wait()

        for b in range(2):
            scan_chunk(bufs[b], b * _RCHUNK, None)
            start_dma(bufs[b], b, sems[b])

        def round_body(g, carry):
            c0 = 2 + g * 2
            for b in range(2):
                c = c0 + b
                r0 = c * _RCHUNK
                wait_dma(bufs[b], sems[b])
                scan_chunk(bufs[b], r0, r0 - 2 * _RCHUNK)
                start_dma(bufs[b], c, sems[b])
            return carry

        lax.fori_loop(0, (_NCHUNK - 3) // 2, round_body, 0)

        c = _NCHUNK - 1
        wait_dma(buf0, sem0)
        scan_chunk(buf0, c * _RCHUNK, (c - 2) * _RCHUNK)
        start_dma(buf0, c, sem0)
        wait_dma(buf1, sem1)
        wait_dma(buf0, sem0)

    return one_hot_kernel


_sc_one_hot = _make_sc_one_hot()


def kernel(labels, src_ones):
    labels_flat = labels.reshape(_B).astype(jnp.int32)
    src_flat = src_ones.reshape(_B).astype(jnp.float32)
    zeros_chunk = jnp.zeros((_RCHUNK, _CPW), jnp.float32)
    out_t = _sc_one_hot(labels_flat, src_flat, zeros_chunk)
    return out_t.T


# speedup vs baseline: 3.2348x; 1.0070x over previous
import functools

import jax
import jax.numpy as jnp
from jax import lax
from jax.experimental import pallas as pl
from jax.experimental.pallas import tpu as pltpu
from jax.experimental.pallas import tpu_sc as plsc

_B = 16384
_N = 1000
_NC = 2
_NS = 16
_CPW = _B // _NS
_RCHUNK = 40
_NGRP = _CPW // 16
_ROWS0 = 440
_CHUNKS0 = _ROWS0 // _RCHUNK
_CHUNKS1 = (_N - _ROWS0) // _RCHUNK


def _make_sc_one_hot():
    mesh = plsc.VectorSubcoreMesh(core_axis_name="c", subcore_axis_name="s")

    @functools.partial(
        pl.kernel,
        out_type=jax.ShapeDtypeStruct((_N, _B), jnp.float32),
        mesh=mesh,
        compiler_params=pltpu.CompilerParams(needs_layout_passes=False),
        scratch_types=[
            pltpu.VMEM((_CPW,), jnp.int32),
            pltpu.VMEM((_CPW,), jnp.float32),
            pltpu.VMEM((_RCHUNK, _CPW), jnp.float32),
            pltpu.VMEM((_RCHUNK, _CPW), jnp.float32),
            pltpu.SemaphoreType.DMA,
            pltpu.SemaphoreType.DMA,
        ],
    )
    def one_hot_kernel(labels_hbm, src_hbm, zeros_hbm, out_hbm,
                       lab_v, src_v, buf0, buf1, sem0, sem1):
        cid = lax.axis_index("c")
        col0 = lax.axis_index("s") * _CPW

        pltpu.sync_copy(labels_hbm.at[pl.ds(col0, _CPW)], lab_v)
        pltpu.sync_copy(src_hbm.at[pl.ds(col0, _CPW)], src_v)
        pltpu.sync_copy(zeros_hbm, buf0)
        pltpu.sync_copy(zeros_hbm, buf1)

        zeros16 = jnp.zeros((16,), jnp.float32)
        iota16 = lax.iota(jnp.int32, 16)
        bufs = (buf0, buf1)
        sems = (sem0, sem1)

        def scan_chunk(buf, new_r0, old_r0):
            def body(g, carry):
                lab16 = lab_v[pl.ds(g * 16, 16)]
                col16 = iota16 + g * 16
                if old_r0 is not None:
                    old_row = lab16 - old_r0
                    old_msk = (old_row >= 0) & (old_row < _RCHUNK)
                    plsc.store_scatter(buf, [old_row, col16], zeros16,
                                       mask=old_msk)
                new_row = lab16 - new_r0
                new_msk = (new_row >= 0) & (new_row < _RCHUNK)
                plsc.store_scatter(buf, [new_row, col16],
                                   src_v[pl.ds(g * 16, 16)], mask=new_msk)
                return carry

            lax.fori_loop(0, _NGRP, body, 0)

        def start_dma(buf, r0, sem):
            dst = out_hbm.at[pl.ds(r0, _RCHUNK), pl.ds(col0, _CPW)]
            pltpu.async_copy(buf, dst, sem)

        def wait_dma(buf, sem):
            pltpu.make_async_copy(
                buf, out_hbm.at[pl.ds(0, _RCHUNK), pl.ds(col0, _CPW)], sem
            ).wait()

        def pipeline(base, nchunk):
            def r0_of(c):
                return base + c * _RCHUNK

            for b in range(2):
                scan_chunk(bufs[b], r0_of(b), None)
                start_dma(bufs[b], r0_of(b), sems[b])

            def round_body(g, carry):
                c0 = 2 + g * 2
                for b in range(2):
                    r0 = r0_of(c0 + b)
                    wait_dma(bufs[b], sems[b])
                    scan_chunk(bufs[b], r0, r0 - 2 * _RCHUNK)
                    start_dma(bufs[b], r0, sems[b])
                return carry

            lax.fori_loop(0, (nchunk - 2) // 2, round_body, 0)

            if nchunk % 2:
                r0 = r0_of(nchunk - 1)
                wait_dma(buf0, sem0)
                scan_chunk(buf0, r0, r0 - 2 * _RCHUNK)
                start_dma(buf0, r0, sem0)
            wait_dma(buf1, sem1)
            wait_dma(buf0, sem0)

        @pl.when(cid == 0)
        def _():
            pipeline(0, _CHUNKS0)

        @pl.when(cid == 1)
        def _():
            pipeline(_ROWS0, _CHUNKS1)

    return one_hot_kernel


_sc_one_hot = _make_sc_one_hot()


def kernel(labels, src_ones):
    labels_flat = labels.reshape(_B).astype(jnp.int32)
    src_flat = src_ones.reshape(_B).astype(jnp.float32)
    zeros_chunk = jnp.zeros((_RCHUNK, _CPW), jnp.float32)
    out_t = _sc_one_hot(labels_flat, src_flat, zeros_chunk)
    return out_t.T
